# baseline (speedup 1.0000x reference)
import functools

import jax
import jax.numpy as jnp
from jax import lax
from jax.experimental import pallas as pl
from jax.experimental.pallas import tpu as pltpu
from jax.experimental.pallas import tpu_sc as plsc

B = 4
L = 4096
D = 1024

B_SC = 1
B_TC = B - B_SC

_info = plsc.get_sparse_core_info()
_NC = _info.num_cores
_NS = _info.num_subcores
_NW = _NC * _NS
_ROWS = L // _NW
_CHUNK = 64
_NCH = _ROWS // _CHUNK

_mesh = plsc.VectorSubcoreMesh(core_axis_name="c", subcore_axis_name="s")


@functools.partial(
    pl.kernel,
    out_type=jax.ShapeDtypeStruct((B_SC * L, D), jnp.float32),
    mesh=_mesh,
    scratch_types=[
        pltpu.VMEM((_CHUNK, D), jnp.float32),
        pltpu.SemaphoreType.DMA,
    ],
)
def _sc_copy(table_hbm, out_hbm, buf, sem):
    wid = lax.axis_index("s") * _NC + lax.axis_index("c")
    base = wid * _ROWS
    for c in range(_NCH):
        off = base + c * _CHUNK
        pltpu.async_copy(table_hbm.at[pl.ds(off, _CHUNK)], buf, sem).wait()
        for b in range(B_SC):
            pltpu.sync_copy(buf, out_hbm.at[pl.ds(b * L + off, _CHUNK)])


_BL = 512


def _tc_body(table_ref, out_ref):
    out_ref[...] = jnp.broadcast_to(table_ref[...][None], (B_TC, _BL, D))


_tc_copy = pl.pallas_call(
    _tc_body,
    grid=(L // _BL,),
    in_specs=[pl.BlockSpec((_BL, D), lambda i: (i, 0))],
    out_specs=pl.BlockSpec((B_TC, _BL, D), lambda i: (0, i, 0)),
    out_shape=jax.ShapeDtypeStruct((B_TC, L, D), jnp.float32),
)


def kernel(x, table):
    del x
    tc_out = _tc_copy(table).reshape(B_TC * L, D)
    sc_out = _sc_copy(table)
    out = jnp.concatenate([tc_out, sc_out], axis=0)
    return out.reshape(B, L, D)


# speedup vs baseline: 1.9975x; 1.9975x over previous
import functools

import jax
import jax.numpy as jnp
from jax import lax
from jax.experimental import pallas as pl
from jax.experimental.pallas import tpu as pltpu
from jax.experimental.pallas import tpu_sc as plsc

B = 4
L = 4096
D = 1024

_info = plsc.get_sparse_core_info()
_NC = _info.num_cores
_NS = _info.num_subcores
_NW = _NC * _NS
_ROWS = L // _NW
_HALF = _ROWS // 2

_mesh = plsc.VectorSubcoreMesh(core_axis_name="c", subcore_axis_name="s")


@functools.partial(
    pl.kernel,
    out_type=jax.ShapeDtypeStruct((B * L, D), jnp.float32),
    mesh=_mesh,
    scratch_types=[
        pltpu.VMEM((_HALF, D), jnp.float32),
        pltpu.VMEM_SHARED((_NS * _HALF, D), jnp.float32),
        pltpu.SemaphoreType.DMA,
        pltpu.SemaphoreType.DMA,
        pltpu.SemaphoreType.DMA,
        pltpu.SemaphoreType.DMA,
    ],
)
def _bcast_copy(table_hbm, out_hbm, tbuf, sbuf, rsem0, rsem1, wsem0, wsem1):
    sid = lax.axis_index("s")
    cid = lax.axis_index("c")
    wid = sid * _NC + cid
    base = wid * _ROWS
    sslice = sbuf.at[pl.ds(sid * _HALF, _HALF)]
    r0 = pltpu.async_copy(table_hbm.at[pl.ds(base, _HALF)], tbuf, rsem0)
    r1 = pltpu.async_copy(table_hbm.at[pl.ds(base + _HALF, _HALF)], sslice,
                          rsem1)
    writes = []
    r0.wait()
    for b in range(B):
        writes.append(pltpu.async_copy(
            tbuf, out_hbm.at[pl.ds(b * L + base, _HALF)], wsem0))
    r1.wait()
    for b in range(B):
        writes.append(pltpu.async_copy(
            sslice, out_hbm.at[pl.ds(b * L + base + _HALF, _HALF)], wsem1))
    for w in writes:
        w.wait()


def kernel(x, table):
    del x
    out = _bcast_copy(table)
    return out.reshape(B, L, D)


# speedup vs baseline: 2.0227x; 1.0126x over previous
import functools

import jax
import jax.numpy as jnp
from jax import lax
from jax.experimental import pallas as pl
from jax.experimental.pallas import tpu as pltpu
from jax.experimental.pallas import tpu_sc as plsc

B = 4
L = 4096
D = 1024

_info = plsc.get_sparse_core_info()
_NC = _info.num_cores
_NS = _info.num_subcores
_NW = _NC * _NS
_ROWS = L // _NW
_CHUNK = 64
_NCH = _ROWS // _CHUNK

_mesh = plsc.VectorSubcoreMesh(core_axis_name="c", subcore_axis_name="s")


@functools.partial(
    pl.kernel,
    out_type=jax.ShapeDtypeStruct((B * L, D), jnp.float32),
    mesh=_mesh,
    scratch_types=[
        pltpu.VMEM((_CHUNK, D), jnp.float32),
        pltpu.SemaphoreType.DMA,
    ],
)
def _bcast_copy(table_hbm, out_hbm, buf, sem):
    wid = lax.axis_index("s") * _NC + lax.axis_index("c")
    base = wid * _ROWS
    for c in range(_NCH):
        off = base + c * _CHUNK
        pltpu.async_copy(table_hbm.at[pl.ds(off, _CHUNK)], buf, sem).wait()
        for b in range(B):
            pltpu.sync_copy(buf, out_hbm.at[pl.ds(b * L + off, _CHUNK)])


def kernel(x, table):
    del x
    out = _bcast_copy(table)
    return out.reshape(B, L, D)
